# Initial kernel scaffold; baseline (speedup 1.0000x reference)
import jax
import jax.numpy as jnp
from jax.experimental import pallas as pl


def kernel(input_vertex_features, input_vertex_coordinates, keypoint_indices, edges, W_off, b_off, W_edge, b_edge, W_upd, b_upd):
    raise NotImplementedError("write your pallas kernel here")


# speedup vs baseline: 3.8740x; 3.8740x over previous
import functools

import jax
import jax.numpy as jnp
from jax import lax
from jax.experimental import pallas as pl
from jax.experimental.pallas import tpu as pltpu
from jax.experimental.pallas import tpu_sc as plsc

N_NODES = 10000
D_FEAT = 300
N_EDGES = 160000
N_KEY = 4096

W = 160
HPAD = 304
ACC_ROWS = 10240
E_PAD = 163840
CH = 128
NCH = 80
NC = 2
NS = 16
ROW_BLK = 1000
ZROWS = ACC_ROWS // NS
CNT_COL = 143


def _offset_body(x_ref, c_ref, w_ref, o_ref):
    o_ref[...] = c_ref[...] + jnp.dot(
        x_ref[...], w_ref[...], preferred_element_type=jnp.float32)


def _tc_offset(x, c4in, w4):
    return pl.pallas_call(
        _offset_body,
        grid=(N_NODES // ROW_BLK,),
        in_specs=[
            pl.BlockSpec((ROW_BLK, D_FEAT), lambda i: (i, 0)),
            pl.BlockSpec((ROW_BLK, 4), lambda i: (i, 0)),
            pl.BlockSpec((D_FEAT, 4), lambda i: (0, 0)),
        ],
        out_specs=pl.BlockSpec((ROW_BLK, 4), lambda i: (i, 0)),
        out_shape=jax.ShapeDtypeStruct((N_NODES, 4), jnp.float32),
    )(x, c4in, w4)


def _final_body(g0_ref, g1_ref, c4_ref, x2_ref, w0_ref, w1_ref, wc_ref,
                wu_ref, o_ref):
    g1 = g1_ref[...]
    cnt = g1[:, CNT_COL:CNT_COL + 1]
    num = (jnp.dot(g0_ref[...], w0_ref[...], preferred_element_type=jnp.float32)
           + jnp.dot(g1, w1_ref[...], preferred_element_type=jnp.float32)
           + cnt * jnp.dot(c4_ref[...], wc_ref[...],
                           preferred_element_type=jnp.float32))
    h_neigh = num / jnp.maximum(cnt, 1.0)
    h = jnp.dot(h_neigh, wu_ref[...],
                preferred_element_type=jnp.float32) + x2_ref[...]
    o_ref[...] = jnp.concatenate(
        [h, jnp.zeros((h.shape[0], HPAD - D_FEAT), jnp.float32)], axis=1)


def _tc_final(g0, g1, c4, x2, w0, w1, wc, wu):
    return pl.pallas_call(
        _final_body,
        grid=(N_NODES // ROW_BLK,),
        in_specs=[
            pl.BlockSpec((ROW_BLK, W), lambda i: (i, 0)),
            pl.BlockSpec((ROW_BLK, W), lambda i: (i, 0)),
            pl.BlockSpec((ROW_BLK, 4), lambda i: (i, 0)),
            pl.BlockSpec((ROW_BLK, D_FEAT), lambda i: (i, 0)),
            pl.BlockSpec((W, D_FEAT), lambda i: (0, 0)),
            pl.BlockSpec((W, D_FEAT), lambda i: (0, 0)),
            pl.BlockSpec((4, D_FEAT), lambda i: (0, 0)),
            pl.BlockSpec((D_FEAT, D_FEAT), lambda i: (0, 0)),
        ],
        out_specs=pl.BlockSpec((ROW_BLK, HPAD), lambda i: (i, 0)),
        out_shape=jax.ShapeDtypeStruct((N_NODES, HPAD), jnp.float32),
    )(g0, g1, c4, x2, w0, w1, wc, wu)


@functools.cache
def _make_sc_segsum():
    return functools.partial(
        pl.kernel,
        out_type=[jax.ShapeDtypeStruct((ACC_ROWS, W), jnp.float32),
                  jax.ShapeDtypeStruct((ACC_ROWS, W), jnp.float32)],
        mesh=plsc.VectorSubcoreMesh(core_axis_name="c", subcore_axis_name="s"),
        scratch_types=[
            pltpu.VMEM((CH,), jnp.int32),
            pltpu.VMEM((CH,), jnp.int32),
            pltpu.VMEM((CH, W), jnp.float32),
            pltpu.VMEM_SHARED((ACC_ROWS, W), jnp.float32),
            pltpu.SemaphoreType.DMA,
        ],
        compiler_params=pltpu.CompilerParams(use_tc_tiling_on_sc=False),
    )(_sc_segsum_body)


def _sc_segsum_body(a0, a1, src_t, dst_t, zrows, g0, g1,
                    sidx, didx, rows, acc, sem):
    c = lax.axis_index("c")
    s = lax.axis_index("s")
    pltpu.sync_copy(zrows, acc.at[pl.ds(s * ZROWS, ZROWS)])
    plsc.subcore_barrier()

    def run(a_ref):
        def body(j, carry):
            pltpu.sync_copy(src_t.at[s, j], sidx)
            pltpu.sync_copy(dst_t.at[s, j], didx)
            pltpu.async_copy(a_ref.at[sidx], rows, sem).wait()
            pltpu.sync_copy(rows, acc.at[didx], add=True)
            return carry
        lax.fori_loop(0, NCH, body, 0)

    @pl.when(c == 0)
    def _():
        run(a0)

    @pl.when(c == 1)
    def _():
        run(a1)

    plsc.subcore_barrier()

    @pl.when(c == 0)
    def _():
        pltpu.sync_copy(acc.at[pl.ds(s * ZROWS, ZROWS)],
                        g0.at[pl.ds(s * ZROWS, ZROWS)])

    @pl.when(c == 1)
    def _():
        pltpu.sync_copy(acc.at[pl.ds(s * ZROWS, ZROWS)],
                        g1.at[pl.ds(s * ZROWS, ZROWS)])


KPW = N_KEY // (NC * NS)


@functools.cache
def _make_sc_kp_gather():
    return functools.partial(
        pl.kernel,
        out_type=jax.ShapeDtypeStruct((N_KEY, HPAD), jnp.float32),
        mesh=plsc.VectorSubcoreMesh(core_axis_name="c", subcore_axis_name="s"),
        scratch_types=[
            pltpu.VMEM((KPW,), jnp.int32),
            pltpu.VMEM((KPW, HPAD), jnp.float32),
            pltpu.SemaphoreType.DMA,
        ],
        compiler_params=pltpu.CompilerParams(use_tc_tiling_on_sc=False),
    )(_sc_kp_gather_body)


def _sc_kp_gather_body(h, kp_t, out, idxv, rows, sem):
    c = lax.axis_index("c")
    s = lax.axis_index("s")
    wid = s * NC + c
    pltpu.sync_copy(kp_t.at[wid], idxv)
    pltpu.async_copy(h.at[idxv], rows, sem).wait()
    pltpu.sync_copy(rows, out.at[pl.ds(wid * KPW, KPW)])


def kernel(input_vertex_features, input_vertex_coordinates, keypoint_indices,
           edges, W_off, b_off, W_edge, b_edge, W_upd, b_upd):
    x = input_vertex_features
    coords = input_vertex_coordinates
    f32 = jnp.float32

    ones = jnp.ones((N_NODES, 1), f32)
    c4in = jnp.concatenate([coords + b_off, ones], axis=1)
    w4 = jnp.concatenate([W_off, jnp.zeros((D_FEAT, 1), f32)], axis=1)

    c4 = _tc_offset(x, c4in, w4)

    npad = W - (D_FEAT - W) - 4
    a0 = x[:, :W]
    a1 = jnp.concatenate(
        [x[:, W:D_FEAT], c4, jnp.zeros((N_NODES, npad), f32)], axis=1)

    src = edges[:, 0].astype(jnp.int32)
    dst = edges[:, 1].astype(jnp.int32)
    pad = E_PAD - N_EDGES
    srcp = jnp.concatenate([src, jnp.zeros((pad,), jnp.int32)])
    dstp = jnp.concatenate([dst, jnp.full((pad,), N_NODES, jnp.int32)])
    src_t = srcp.reshape(NS, NCH, CH)
    dst_t = dstp.reshape(NS, NCH, CH)
    zrows = jnp.zeros((ZROWS, W), f32)

    g0, g1 = _make_sc_segsum()(a0, a1, src_t, dst_t, zrows)

    w0 = W_edge[:W]
    w1 = jnp.concatenate(
        [W_edge[W:D_FEAT], -W_edge[D_FEAT:D_FEAT + 3], b_edge[None, :],
         jnp.zeros((npad, D_FEAT), f32)], axis=0)
    wc = jnp.concatenate([W_edge[D_FEAT:D_FEAT + 3],
                          jnp.zeros((1, D_FEAT), f32)], axis=0)
    x2 = x + b_upd

    h = _tc_final(g0, g1, c4, x2, w0, w1, wc, W_upd)

    kp_t = keypoint_indices[:, 0].astype(jnp.int32).reshape(NC * NS, KPW)
    out = _make_sc_kp_gather()(h, kp_t)
    return out[:, :D_FEAT]


# speedup vs baseline: 4.3980x; 1.1353x over previous
import functools

import jax
import jax.numpy as jnp
from jax import lax
from jax.experimental import pallas as pl
from jax.experimental.pallas import tpu as pltpu
from jax.experimental.pallas import tpu_sc as plsc

N_NODES = 10000
D_FEAT = 300
N_EDGES = 160000
N_KEY = 4096

W = 160
HPAD = 304
ACC_ROWS = 10240
E_PAD = 163840
CH = 64
NCH = 160
NC = 2
NS = 16
ROW_BLK = 1000
KB = 8
ZROWS = ACC_ROWS // NS
CNT_COL = 143


def _offset_body(x_ref, c_ref, w_ref, o_ref):
    o_ref[...] = c_ref[...] + jnp.dot(
        x_ref[...], w_ref[...], preferred_element_type=jnp.float32)


def _tc_offset(x, c4in, w4):
    return pl.pallas_call(
        _offset_body,
        grid=(N_NODES // ROW_BLK,),
        in_specs=[
            pl.BlockSpec((ROW_BLK, D_FEAT), lambda i: (i, 0)),
            pl.BlockSpec((ROW_BLK, 4), lambda i: (i, 0)),
            pl.BlockSpec((D_FEAT, 4), lambda i: (0, 0)),
        ],
        out_specs=pl.BlockSpec((ROW_BLK, 4), lambda i: (i, 0)),
        out_shape=jax.ShapeDtypeStruct((N_NODES, 4), jnp.float32),
    )(x, c4in, w4)


def _final_body(g0_ref, g1_ref, c4_ref, x2_ref, w0_ref, w1_ref, wc_ref,
                wu_ref, o_ref):
    g1 = g1_ref[...]
    cnt = g1[:, CNT_COL:CNT_COL + 1]
    num = (jnp.dot(g0_ref[...], w0_ref[...], preferred_element_type=jnp.float32)
           + jnp.dot(g1, w1_ref[...], preferred_element_type=jnp.float32)
           + cnt * jnp.dot(c4_ref[...], wc_ref[...],
                           preferred_element_type=jnp.float32))
    h_neigh = num / jnp.maximum(cnt, 1.0)
    h = jnp.dot(h_neigh, wu_ref[...],
                preferred_element_type=jnp.float32) + x2_ref[...]
    o_ref[...] = jnp.concatenate(
        [h, jnp.zeros((h.shape[0], HPAD - D_FEAT), jnp.float32)], axis=1)


def _tc_final(g0, g1, c4, x2, w0, w1, wc, wu):
    return pl.pallas_call(
        _final_body,
        grid=(N_NODES // ROW_BLK,),
        in_specs=[
            pl.BlockSpec((ROW_BLK, W), lambda i: (i, 0)),
            pl.BlockSpec((ROW_BLK, W), lambda i: (i, 0)),
            pl.BlockSpec((ROW_BLK, 4), lambda i: (i, 0)),
            pl.BlockSpec((ROW_BLK, D_FEAT), lambda i: (i, 0)),
            pl.BlockSpec((W, D_FEAT), lambda i: (0, 0)),
            pl.BlockSpec((W, D_FEAT), lambda i: (0, 0)),
            pl.BlockSpec((4, D_FEAT), lambda i: (0, 0)),
            pl.BlockSpec((D_FEAT, D_FEAT), lambda i: (0, 0)),
        ],
        out_specs=pl.BlockSpec((ROW_BLK, HPAD), lambda i: (i, 0)),
        out_shape=jax.ShapeDtypeStruct((N_NODES, HPAD), jnp.float32),
    )(g0, g1, c4, x2, w0, w1, wc, wu)


@functools.cache
def _make_sc_segsum():
    return functools.partial(
        pl.kernel,
        out_type=[jax.ShapeDtypeStruct((ACC_ROWS, W), jnp.float32),
                  jax.ShapeDtypeStruct((ACC_ROWS, W), jnp.float32)],
        mesh=plsc.VectorSubcoreMesh(core_axis_name="c", subcore_axis_name="s"),
        scratch_types=[
            pltpu.VMEM((2, KB, CH), jnp.int32),
            pltpu.VMEM((2, KB, CH), jnp.int32),
            pltpu.VMEM((2, CH, W), jnp.float32),
            pltpu.VMEM_SHARED((ACC_ROWS, W), jnp.float32),
            pltpu.SemaphoreType.DMA,
            pltpu.SemaphoreType.DMA,
            pltpu.SemaphoreType.DMA,
        ],
        compiler_params=pltpu.CompilerParams(use_tc_tiling_on_sc=False),
    )(_sc_segsum_body)


def _sc_segsum_body(a0, a1, src_t, dst_t, zrows, g0, g1,
                    sidx, didx, rows, acc, sem_g, sem_s, sem_i):
    c = lax.axis_index("c")
    s = lax.axis_index("s")
    NB = NCH // KB
    pltpu.sync_copy(zrows, acc.at[pl.ds(s * ZROWS, ZROWS)])
    plsc.subcore_barrier()

    def idx_fetch(t, bb):
        for k in range(KB):
            pltpu.async_copy(src_t.at[s, t * KB + k], sidx.at[bb, k], sem_i)
            pltpu.async_copy(dst_t.at[s, t * KB + k], didx.at[bb, k], sem_i)

    def idx_drain(t, bb):
        for k in range(KB):
            pltpu.make_async_copy(
                src_t.at[s, t * KB + k], sidx.at[bb, k], sem_i).wait()
            pltpu.make_async_copy(
                dst_t.at[s, t * KB + k], didx.at[bb, k], sem_i).wait()

    def run(a_ref):
        idx_fetch(0, 0)

        def block(t, carry):
            bb = lax.rem(t, 2)
            idx_drain(t, bb)

            @pl.when(t + 1 < NB)
            def _():
                idx_fetch(t + 1, 1 - bb)

            gathers = [
                pltpu.make_async_copy(a_ref.at[sidx.at[bb, k]],
                                      rows.at[k % 2], sem_g)
                for k in range(KB)
            ]
            scatters = [
                pltpu.make_async_copy(rows.at[k % 2],
                                      acc.at[didx.at[bb, k]], sem_s)
                for k in range(KB)
            ]
            gathers[0].start()
            for k in range(KB):
                gathers[k].wait()
                if k >= 1:
                    scatters[k - 1].wait()
                if k < KB - 1:
                    gathers[k + 1].start()
                scatters[k].start(add=True)
            scatters[KB - 1].wait()
            return carry
        lax.fori_loop(0, NB, block, 0)

    @pl.when(c == 0)
    def _():
        run(a0)

    @pl.when(c == 1)
    def _():
        run(a1)

    plsc.subcore_barrier()

    @pl.when(c == 0)
    def _():
        pltpu.sync_copy(acc.at[pl.ds(s * ZROWS, ZROWS)],
                        g0.at[pl.ds(s * ZROWS, ZROWS)])

    @pl.when(c == 1)
    def _():
        pltpu.sync_copy(acc.at[pl.ds(s * ZROWS, ZROWS)],
                        g1.at[pl.ds(s * ZROWS, ZROWS)])


KPW = N_KEY // (NC * NS)


@functools.cache
def _make_sc_kp_gather():
    return functools.partial(
        pl.kernel,
        out_type=jax.ShapeDtypeStruct((N_KEY, HPAD), jnp.float32),
        mesh=plsc.VectorSubcoreMesh(core_axis_name="c", subcore_axis_name="s"),
        scratch_types=[
            pltpu.VMEM((KPW,), jnp.int32),
            pltpu.VMEM((KPW, HPAD), jnp.float32),
            pltpu.SemaphoreType.DMA,
        ],
        compiler_params=pltpu.CompilerParams(use_tc_tiling_on_sc=False),
    )(_sc_kp_gather_body)


def _sc_kp_gather_body(h, kp_t, out, idxv, rows, sem):
    c = lax.axis_index("c")
    s = lax.axis_index("s")
    wid = s * NC + c
    pltpu.sync_copy(kp_t.at[wid], idxv)
    pltpu.async_copy(h.at[idxv], rows, sem).wait()
    pltpu.sync_copy(rows, out.at[pl.ds(wid * KPW, KPW)])


def kernel(input_vertex_features, input_vertex_coordinates, keypoint_indices,
           edges, W_off, b_off, W_edge, b_edge, W_upd, b_upd):
    x = input_vertex_features
    coords = input_vertex_coordinates
    f32 = jnp.float32

    ones = jnp.ones((N_NODES, 1), f32)
    c4in = jnp.concatenate([coords + b_off, ones], axis=1)
    w4 = jnp.concatenate([W_off, jnp.zeros((D_FEAT, 1), f32)], axis=1)

    c4 = _tc_offset(x, c4in, w4)

    npad = W - (D_FEAT - W) - 4
    a0 = x[:, :W]
    a1 = jnp.concatenate(
        [x[:, W:D_FEAT], c4, jnp.zeros((N_NODES, npad), f32)], axis=1)

    src = edges[:, 0].astype(jnp.int32)
    dst = edges[:, 1].astype(jnp.int32)
    pad = E_PAD - N_EDGES
    srcp = jnp.concatenate([src, jnp.zeros((pad,), jnp.int32)])
    dstp = jnp.concatenate([dst, jnp.full((pad,), N_NODES, jnp.int32)])
    src_t = srcp.reshape(NS, NCH, CH)
    dst_t = dstp.reshape(NS, NCH, CH)
    zrows = jnp.zeros((ZROWS, W), f32)

    g0, g1 = _make_sc_segsum()(a0, a1, src_t, dst_t, zrows)

    w0 = W_edge[:W]
    w1 = jnp.concatenate(
        [W_edge[W:D_FEAT], -W_edge[D_FEAT:D_FEAT + 3], b_edge[None, :],
         jnp.zeros((npad, D_FEAT), f32)], axis=0)
    wc = jnp.concatenate([W_edge[D_FEAT:D_FEAT + 3],
                          jnp.zeros((1, D_FEAT), f32)], axis=0)
    x2 = x + b_upd

    h = _tc_final(g0, g1, c4, x2, w0, w1, wc, W_upd)

    kp_t = keypoint_indices[:, 0].astype(jnp.int32).reshape(NC * NS, KPW)
    out = _make_sc_kp_gather()(h, kp_t)
    return out[:, :D_FEAT]


# speedup vs baseline: 5.3588x; 1.2184x over previous
import functools

import jax
import jax.numpy as jnp
from jax import lax
from jax.experimental import pallas as pl
from jax.experimental.pallas import tpu as pltpu
from jax.experimental.pallas import tpu_sc as plsc

N_NODES = 10000
D_FEAT = 300
N_EDGES = 160000
N_KEY = 4096

W = 160
HPAD = 304
ACC_ROWS = 10240
E_PAD = 163840
CH = 128
NCH = 80
NC = 2
NS = 16
ROW_BLK = 1000
KB = 8
ZROWS = ACC_ROWS // NS
CNT_COL = 143


def _offset_body(x_ref, c_ref, w_ref, o_ref, a0_ref, a1_ref):
    x = x_ref[...]
    c4 = c_ref[...] + jnp.dot(x, w_ref[...], preferred_element_type=jnp.float32)
    o_ref[...] = c4
    a0_ref[...] = x[:, :W].astype(jnp.bfloat16)
    npad = W - (D_FEAT - W) - 4
    a1_ref[...] = jnp.concatenate(
        [x[:, W:D_FEAT], c4, jnp.zeros((x.shape[0], npad), jnp.float32)],
        axis=1).astype(jnp.bfloat16)


def _tc_offset(x, c4in, w4):
    return pl.pallas_call(
        _offset_body,
        grid=(N_NODES // ROW_BLK,),
        in_specs=[
            pl.BlockSpec((ROW_BLK, D_FEAT), lambda i: (i, 0)),
            pl.BlockSpec((ROW_BLK, 4), lambda i: (i, 0)),
            pl.BlockSpec((D_FEAT, 4), lambda i: (0, 0)),
        ],
        out_specs=[pl.BlockSpec((ROW_BLK, 4), lambda i: (i, 0)),
                   pl.BlockSpec((ROW_BLK, W), lambda i: (i, 0)),
                   pl.BlockSpec((ROW_BLK, W), lambda i: (i, 0))],
        out_shape=[jax.ShapeDtypeStruct((N_NODES, 4), jnp.float32),
                   jax.ShapeDtypeStruct((N_NODES, W), jnp.bfloat16),
                   jax.ShapeDtypeStruct((N_NODES, W), jnp.bfloat16)],
    )(x, c4in, w4)


def _final_body(g0_ref, g1_ref, c4_ref, x2_ref, w0_ref, w1_ref, wc_ref,
                wu_ref, o_ref):
    g1 = g1_ref[...].astype(jnp.float32)
    g0 = g0_ref[...].astype(jnp.float32)
    cnt = g1[:, CNT_COL:CNT_COL + 1]
    num = (jnp.dot(g0, w0_ref[...], preferred_element_type=jnp.float32)
           + jnp.dot(g1, w1_ref[...], preferred_element_type=jnp.float32)
           + cnt * jnp.dot(c4_ref[...], wc_ref[...],
                           preferred_element_type=jnp.float32))
    h_neigh = num / jnp.maximum(cnt, 1.0)
    h = jnp.dot(h_neigh, wu_ref[...],
                preferred_element_type=jnp.float32) + x2_ref[...]
    o_ref[...] = jnp.concatenate(
        [h, jnp.zeros((h.shape[0], HPAD - D_FEAT), jnp.float32)], axis=1)


def _tc_final(g0, g1, c4, x2, w0, w1, wc, wu):
    return pl.pallas_call(
        _final_body,
        grid=(N_NODES // ROW_BLK,),
        in_specs=[
            pl.BlockSpec((ROW_BLK, W), lambda i: (i, 0)),
            pl.BlockSpec((ROW_BLK, W), lambda i: (i, 0)),
            pl.BlockSpec((ROW_BLK, 4), lambda i: (i, 0)),
            pl.BlockSpec((ROW_BLK, D_FEAT), lambda i: (i, 0)),
            pl.BlockSpec((W, D_FEAT), lambda i: (0, 0)),
            pl.BlockSpec((W, D_FEAT), lambda i: (0, 0)),
            pl.BlockSpec((4, D_FEAT), lambda i: (0, 0)),
            pl.BlockSpec((D_FEAT, D_FEAT), lambda i: (0, 0)),
        ],
        out_specs=pl.BlockSpec((ROW_BLK, HPAD), lambda i: (i, 0)),
        out_shape=jax.ShapeDtypeStruct((N_NODES, HPAD), jnp.float32),
    )(g0, g1, c4, x2, w0, w1, wc, wu)


@functools.cache
def _make_sc_segsum():
    return functools.partial(
        pl.kernel,
        out_type=[jax.ShapeDtypeStruct((ACC_ROWS, W), jnp.bfloat16),
                  jax.ShapeDtypeStruct((ACC_ROWS, W), jnp.bfloat16)],
        mesh=plsc.VectorSubcoreMesh(core_axis_name="c", subcore_axis_name="s"),
        scratch_types=[
            pltpu.VMEM((2, KB, CH), jnp.int32),
            pltpu.VMEM((2, KB, CH), jnp.int32),
            pltpu.VMEM((2, CH, W), jnp.bfloat16),
            pltpu.VMEM_SHARED((ACC_ROWS, W), jnp.bfloat16),
            pltpu.SemaphoreType.DMA,
            pltpu.SemaphoreType.DMA,
            pltpu.SemaphoreType.DMA,
        ],
        compiler_params=pltpu.CompilerParams(use_tc_tiling_on_sc=False),
    )(_sc_segsum_body)


def _sc_segsum_body(a0, a1, src_t, dst_t, zrows, g0, g1,
                    sidx, didx, rows, acc, sem_g, sem_s, sem_i):
    c = lax.axis_index("c")
    s = lax.axis_index("s")
    NB = NCH // KB
    pltpu.sync_copy(zrows, acc.at[pl.ds(s * ZROWS, ZROWS)])
    plsc.subcore_barrier()

    def idx_fetch(t, bb):
        for k in range(KB):
            pltpu.async_copy(src_t.at[s, t * KB + k], sidx.at[bb, k], sem_i)
            pltpu.async_copy(dst_t.at[s, t * KB + k], didx.at[bb, k], sem_i)

    def idx_drain(t, bb):
        for k in range(KB):
            pltpu.make_async_copy(
                src_t.at[s, t * KB + k], sidx.at[bb, k], sem_i).wait()
            pltpu.make_async_copy(
                dst_t.at[s, t * KB + k], didx.at[bb, k], sem_i).wait()

    def run(a_ref):
        idx_fetch(0, 0)

        def block(t, carry):
            bb = lax.rem(t, 2)
            idx_drain(t, bb)

            @pl.when(t + 1 < NB)
            def _():
                idx_fetch(t + 1, 1 - bb)

            gathers = [
                pltpu.make_async_copy(a_ref.at[sidx.at[bb, k]],
                                      rows.at[k % 2], sem_g)
                for k in range(KB)
            ]
            scatters = [
                pltpu.make_async_copy(rows.at[k % 2],
                                      acc.at[didx.at[bb, k]], sem_s)
                for k in range(KB)
            ]
            gathers[0].start()
            for k in range(KB):
                gathers[k].wait()
                if k >= 1:
                    scatters[k - 1].wait()
                if k < KB - 1:
                    gathers[k + 1].start()
                scatters[k].start(add=True)
            scatters[KB - 1].wait()
            return carry
        lax.fori_loop(0, NB, block, 0)

    @pl.when(c == 0)
    def _():
        run(a0)

    @pl.when(c == 1)
    def _():
        run(a1)

    plsc.subcore_barrier()

    @pl.when(c == 0)
    def _():
        pltpu.sync_copy(acc.at[pl.ds(s * ZROWS, ZROWS)],
                        g0.at[pl.ds(s * ZROWS, ZROWS)])

    @pl.when(c == 1)
    def _():
        pltpu.sync_copy(acc.at[pl.ds(s * ZROWS, ZROWS)],
                        g1.at[pl.ds(s * ZROWS, ZROWS)])


KPW = N_KEY // (NC * NS)


@functools.cache
def _make_sc_kp_gather():
    return functools.partial(
        pl.kernel,
        out_type=jax.ShapeDtypeStruct((N_KEY, HPAD), jnp.float32),
        mesh=plsc.VectorSubcoreMesh(core_axis_name="c", subcore_axis_name="s"),
        scratch_types=[
            pltpu.VMEM((KPW,), jnp.int32),
            pltpu.VMEM((KPW, HPAD), jnp.float32),
            pltpu.SemaphoreType.DMA,
        ],
        compiler_params=pltpu.CompilerParams(use_tc_tiling_on_sc=False),
    )(_sc_kp_gather_body)


def _sc_kp_gather_body(h, kp_t, out, idxv, rows, sem):
    c = lax.axis_index("c")
    s = lax.axis_index("s")
    wid = s * NC + c
    pltpu.sync_copy(kp_t.at[wid], idxv)
    pltpu.async_copy(h.at[idxv], rows, sem).wait()
    pltpu.sync_copy(rows, out.at[pl.ds(wid * KPW, KPW)])


def kernel(input_vertex_features, input_vertex_coordinates, keypoint_indices,
           edges, W_off, b_off, W_edge, b_edge, W_upd, b_upd):
    x = input_vertex_features
    coords = input_vertex_coordinates
    f32 = jnp.float32

    ones = jnp.ones((N_NODES, 1), f32)
    c4in = jnp.concatenate([coords + b_off, ones], axis=1)
    w4 = jnp.concatenate([W_off, jnp.zeros((D_FEAT, 1), f32)], axis=1)

    c4, a0, a1 = _tc_offset(x, c4in, w4)
    npad = W - (D_FEAT - W) - 4

    src = edges[:, 0].astype(jnp.int32)
    dst = edges[:, 1].astype(jnp.int32)
    pad = E_PAD - N_EDGES
    srcp = jnp.concatenate([src, jnp.zeros((pad,), jnp.int32)])
    dstp = jnp.concatenate([dst, jnp.full((pad,), N_NODES, jnp.int32)])
    src_t = srcp.reshape(NS, NCH, CH)
    dst_t = dstp.reshape(NS, NCH, CH)
    zrows = jnp.zeros((ZROWS, W), jnp.bfloat16)

    g0, g1 = _make_sc_segsum()(a0, a1, src_t, dst_t, zrows)

    w0 = W_edge[:W]
    w1 = jnp.concatenate(
        [W_edge[W:D_FEAT], -W_edge[D_FEAT:D_FEAT + 3], b_edge[None, :],
         jnp.zeros((npad, D_FEAT), f32)], axis=0)
    wc = jnp.concatenate([W_edge[D_FEAT:D_FEAT + 3],
                          jnp.zeros((1, D_FEAT), f32)], axis=0)
    x2 = x + b_upd

    h = _tc_final(g0, g1, c4, x2, w0, w1, wc, W_upd)

    kp_t = keypoint_indices[:, 0].astype(jnp.int32).reshape(NC * NS, KPW)
    out = _make_sc_kp_gather()(h, kp_t)
    return out[:, :D_FEAT]


# speedup vs baseline: 5.9788x; 1.1157x over previous
import functools

import jax
import jax.numpy as jnp
from jax import lax
from jax.experimental import pallas as pl
from jax.experimental.pallas import tpu as pltpu
from jax.experimental.pallas import tpu_sc as plsc

N_NODES = 10000
D_FEAT = 300
N_EDGES = 160000
N_KEY = 4096

W = 160
XPAD = 304
ACC_ROWS = 10240
E_PAD = 163840
CH = 128
NCH = 80
NC = 2
NS = 16
ROW_BLK = 1000
KB = 16
ZROWS = ACC_ROWS // NS
CNT_COL = 143
KPT = N_KEY // NS
KROW_BLK = 1024


def _offset_body(x_ref, c_ref, w_ref, c16_ref, a0_ref, a1_ref, xp_ref):
    x = x_ref[...]
    c16 = c_ref[...] + jnp.dot(x, w_ref[...],
                               preferred_element_type=jnp.float32)
    c16_ref[...] = c16
    a0_ref[...] = x[:, :W].astype(jnp.bfloat16)
    npad = W - (D_FEAT - W) - 4
    a1_ref[...] = jnp.concatenate(
        [x[:, W:D_FEAT], c16[:, :4],
         jnp.zeros((x.shape[0], npad), jnp.float32)],
        axis=1).astype(jnp.bfloat16)
    xp_ref[...] = jnp.concatenate(
        [x, jnp.zeros((x.shape[0], XPAD - D_FEAT), jnp.float32)], axis=1)


def _tc_offset(x, c16in, w16):
    return pl.pallas_call(
        _offset_body,
        grid=(N_NODES // ROW_BLK,),
        in_specs=[
            pl.BlockSpec((ROW_BLK, D_FEAT), lambda i: (i, 0)),
            pl.BlockSpec((ROW_BLK, 16), lambda i: (i, 0)),
            pl.BlockSpec((D_FEAT, 16), lambda i: (0, 0)),
        ],
        out_specs=[pl.BlockSpec((ROW_BLK, 16), lambda i: (i, 0)),
                   pl.BlockSpec((ROW_BLK, W), lambda i: (i, 0)),
                   pl.BlockSpec((ROW_BLK, W), lambda i: (i, 0)),
                   pl.BlockSpec((ROW_BLK, XPAD), lambda i: (i, 0))],
        out_shape=[jax.ShapeDtypeStruct((N_NODES, 16), jnp.float32),
                   jax.ShapeDtypeStruct((N_NODES, W), jnp.bfloat16),
                   jax.ShapeDtypeStruct((N_NODES, W), jnp.bfloat16),
                   jax.ShapeDtypeStruct((N_NODES, XPAD), jnp.float32)],
    )(x, c16in, w16)


def _final_body(g0_ref, g1_ref, c16_ref, xk_ref, w0_ref, w1_ref, wc_ref,
                wu_ref, o_ref):
    g0 = g0_ref[...].astype(jnp.float32)
    g1 = g1_ref[...].astype(jnp.float32)
    cnt = g1[:, CNT_COL:CNT_COL + 1]
    num = (jnp.dot(g0, w0_ref[...], preferred_element_type=jnp.float32)
           + jnp.dot(g1, w1_ref[...], preferred_element_type=jnp.float32)
           + cnt * jnp.dot(c16_ref[...], wc_ref[...],
                           preferred_element_type=jnp.float32))
    hn = num / jnp.maximum(cnt, 1.0)
    n = hn.shape[0]
    hn2 = jnp.concatenate(
        [hn, jnp.ones((n, 1), jnp.float32), jnp.zeros((n, 3), jnp.float32)],
        axis=1)
    o_ref[...] = jnp.dot(hn2, wu_ref[...],
                         preferred_element_type=jnp.float32) \
        + xk_ref[...][:, :D_FEAT]


def _tc_final(g0k, g1k, c16k, xk, w0, w1, wc, wu2):
    return pl.pallas_call(
        _final_body,
        grid=(N_KEY // KROW_BLK,),
        in_specs=[
            pl.BlockSpec((KROW_BLK, W), lambda i: (i, 0)),
            pl.BlockSpec((KROW_BLK, W), lambda i: (i, 0)),
            pl.BlockSpec((KROW_BLK, 16), lambda i: (i, 0)),
            pl.BlockSpec((KROW_BLK, XPAD), lambda i: (i, 0)),
            pl.BlockSpec((W, D_FEAT), lambda i: (0, 0)),
            pl.BlockSpec((W, D_FEAT), lambda i: (0, 0)),
            pl.BlockSpec((16, D_FEAT), lambda i: (0, 0)),
            pl.BlockSpec((XPAD, D_FEAT), lambda i: (0, 0)),
        ],
        out_specs=pl.BlockSpec((KROW_BLK, D_FEAT), lambda i: (i, 0)),
        out_shape=jax.ShapeDtypeStruct((N_KEY, D_FEAT), jnp.float32),
    )(g0k, g1k, c16k, xk, w0, w1, wc, wu2)


@functools.cache
def _make_sc_segsum():
    return functools.partial(
        pl.kernel,
        out_type=[jax.ShapeDtypeStruct((N_KEY, W), jnp.bfloat16),
                  jax.ShapeDtypeStruct((N_KEY, W), jnp.bfloat16),
                  jax.ShapeDtypeStruct((N_KEY, XPAD), jnp.float32),
                  jax.ShapeDtypeStruct((N_KEY, 16), jnp.float32)],
        mesh=plsc.VectorSubcoreMesh(core_axis_name="c", subcore_axis_name="s"),
        scratch_types=[
            pltpu.VMEM((2, KB, CH), jnp.int32),
            pltpu.VMEM((2, KB, CH), jnp.int32),
            pltpu.VMEM((3, CH, W), jnp.bfloat16),
            pltpu.VMEM_SHARED((ACC_ROWS, W), jnp.bfloat16),
            pltpu.VMEM((CH,), jnp.int32),
            pltpu.VMEM((CH, W), jnp.bfloat16),
            pltpu.VMEM((64, XPAD), jnp.float32),
            pltpu.VMEM((CH, 16), jnp.float32),
            pltpu.SemaphoreType.DMA,
            pltpu.SemaphoreType.DMA,
            pltpu.SemaphoreType.DMA,
        ],
        compiler_params=pltpu.CompilerParams(use_tc_tiling_on_sc=False),
    )(_sc_segsum_body)


def _sc_segsum_body(a0, a1, src_t, dst_t, zrows, kp_t, xpad, c16,
                    g0k, g1k, xk, c16k,
                    sidx, didx, rows, acc, kidx, kbuf, xbuf, cbuf,
                    sem_g, sem_s, sem_i):
    c = lax.axis_index("c")
    s = lax.axis_index("s")
    NB = NCH // KB
    pltpu.sync_copy(zrows, acc.at[pl.ds(s * ZROWS, ZROWS)])
    plsc.subcore_barrier()

    def idx_fetch(t, bb):
        for k in range(KB):
            pltpu.async_copy(src_t.at[s, t * KB + k], sidx.at[bb, k], sem_i)
            pltpu.async_copy(dst_t.at[s, t * KB + k], didx.at[bb, k], sem_i)

    def idx_drain(t, bb):
        for k in range(KB):
            pltpu.make_async_copy(
                src_t.at[s, t * KB + k], sidx.at[bb, k], sem_i).wait()
            pltpu.make_async_copy(
                dst_t.at[s, t * KB + k], didx.at[bb, k], sem_i).wait()

    def run(a_ref):
        idx_fetch(0, 0)

        def block(t, carry):
            bb = lax.rem(t, 2)
            idx_drain(t, bb)

            @pl.when(t + 1 < NB)
            def _():
                idx_fetch(t + 1, 1 - bb)

            gathers = [
                pltpu.make_async_copy(a_ref.at[sidx.at[bb, k]],
                                      rows.at[k % 3], sem_g)
                for k in range(KB)
            ]
            scatters = [
                pltpu.make_async_copy(rows.at[k % 3],
                                      acc.at[didx.at[bb, k]], sem_s)
                for k in range(KB)
            ]
            gathers[0].start()
            gathers[1].start()
            for k in range(KB):
                gathers[k].wait()
                if k >= 1:
                    scatters[k - 1].wait()
                if k + 2 < KB:
                    gathers[k + 2].start()
                scatters[k].start(add=True)
            scatters[KB - 1].wait()
            return carry
        lax.fori_loop(0, NB, block, 0)

    @pl.when(c == 0)
    def _():
        run(a0)

    @pl.when(c == 1)
    def _():
        run(a1)

    plsc.subcore_barrier()

    @pl.when(c == 0)
    def _():
        for q in range(2):
            pltpu.sync_copy(kp_t.at[s, q], kidx)
            pltpu.async_copy(acc.at[kidx], kbuf, sem_g).wait()
            pltpu.sync_copy(kbuf, g0k.at[pl.ds(s * KPT + q * CH, CH)])
            for i in range(2):
                pltpu.async_copy(
                    xpad.at[kidx.at[pl.ds(i * 64, 64)]], xbuf, sem_g).wait()
                pltpu.sync_copy(
                    xbuf, xk.at[pl.ds(s * KPT + q * CH + i * 64, 64)])

    @pl.when(c == 1)
    def _():
        for q in range(2):
            pltpu.sync_copy(kp_t.at[s, q], kidx)
            pltpu.async_copy(acc.at[kidx], kbuf, sem_g).wait()
            pltpu.sync_copy(kbuf, g1k.at[pl.ds(s * KPT + q * CH, CH)])
            pltpu.async_copy(c16.at[kidx], cbuf, sem_g).wait()
            pltpu.sync_copy(cbuf, c16k.at[pl.ds(s * KPT + q * CH, CH)])


def kernel(input_vertex_features, input_vertex_coordinates, keypoint_indices,
           edges, W_off, b_off, W_edge, b_edge, W_upd, b_upd):
    x = input_vertex_features
    coords = input_vertex_coordinates
    f32 = jnp.float32

    ones = jnp.ones((N_NODES, 1), f32)
    c16in = jnp.concatenate(
        [coords + b_off, ones, jnp.zeros((N_NODES, 12), f32)], axis=1)
    w16 = jnp.concatenate([W_off, jnp.zeros((D_FEAT, 13), f32)], axis=1)

    c16, a0, a1, xpad = _tc_offset(x, c16in, w16)

    src = edges[:, 0].astype(jnp.int32)
    dst = edges[:, 1].astype(jnp.int32)
    pad = E_PAD - N_EDGES
    srcp = jnp.concatenate([src, jnp.zeros((pad,), jnp.int32)])
    dstp = jnp.concatenate([dst, jnp.full((pad,), N_NODES, jnp.int32)])
    src_t = srcp.reshape(NS, NCH, CH)
    dst_t = dstp.reshape(NS, NCH, CH)
    zrows = jnp.zeros((ZROWS, W), jnp.bfloat16)
    kp_t = keypoint_indices[:, 0].astype(jnp.int32).reshape(NS, 2, CH)

    g0k, g1k, xk, c16k = _make_sc_segsum()(
        a0, a1, src_t, dst_t, zrows, kp_t, xpad, c16)

    npad = W - (D_FEAT - W) - 4
    w0 = W_edge[:W]
    w1 = jnp.concatenate(
        [W_edge[W:D_FEAT], -W_edge[D_FEAT:D_FEAT + 3], b_edge[None, :],
         jnp.zeros((npad, D_FEAT), f32)], axis=0)
    wc16 = jnp.concatenate([W_edge[D_FEAT:D_FEAT + 3],
                            jnp.zeros((13, D_FEAT), f32)], axis=0)
    wu2 = jnp.concatenate(
        [W_upd, b_upd[None, :], jnp.zeros((3, D_FEAT), f32)], axis=0)

    return _tc_final(g0k, g1k, c16k, xk, w0, w1, wc16, wu2)


# speedup vs baseline: 6.2448x; 1.0445x over previous
import functools

import jax
import jax.numpy as jnp
from jax import lax
from jax.experimental import pallas as pl
from jax.experimental.pallas import tpu as pltpu
from jax.experimental.pallas import tpu_sc as plsc

N_NODES = 10000
D_FEAT = 300
N_EDGES = 160000
N_KEY = 4096

W = 160
XPAD = 304
ACC_ROWS = 10240
E_PAD = 163840
CH = 128
NCH = 80
NC = 2
NS = 16
ROW_BLK = 1000
KB = 16
ZROWS = ACC_ROWS // NS
CNT_COL = 143
KPT = N_KEY // NS
KROW_BLK = 1024


def _offset_body(x_ref, c_ref, w_ref, c16_ref, a0_ref, a1_ref, xp_ref):
    x = x_ref[...]
    c16 = c_ref[...] + jnp.dot(x, w_ref[...],
                               preferred_element_type=jnp.float32)
    c16_ref[...] = c16
    a0_ref[...] = x[:, :W].astype(jnp.bfloat16)
    npad = W - (D_FEAT - W) - 4
    a1_ref[...] = jnp.concatenate(
        [x[:, W:D_FEAT], c16[:, :4],
         jnp.zeros((x.shape[0], npad), jnp.float32)],
        axis=1).astype(jnp.bfloat16)
    xp_ref[...] = jnp.concatenate(
        [x, jnp.zeros((x.shape[0], XPAD - D_FEAT), jnp.float32)], axis=1)


def _tc_offset(x, c16in, w16):
    return pl.pallas_call(
        _offset_body,
        grid=(N_NODES // ROW_BLK,),
        in_specs=[
            pl.BlockSpec((ROW_BLK, D_FEAT), lambda i: (i, 0)),
            pl.BlockSpec((ROW_BLK, 16), lambda i: (i, 0)),
            pl.BlockSpec((D_FEAT, 16), lambda i: (0, 0)),
        ],
        out_specs=[pl.BlockSpec((ROW_BLK, 16), lambda i: (i, 0)),
                   pl.BlockSpec((ROW_BLK, W), lambda i: (i, 0)),
                   pl.BlockSpec((ROW_BLK, W), lambda i: (i, 0)),
                   pl.BlockSpec((ROW_BLK, XPAD), lambda i: (i, 0))],
        out_shape=[jax.ShapeDtypeStruct((N_NODES, 16), jnp.float32),
                   jax.ShapeDtypeStruct((N_NODES, W), jnp.bfloat16),
                   jax.ShapeDtypeStruct((N_NODES, W), jnp.bfloat16),
                   jax.ShapeDtypeStruct((N_NODES, XPAD), jnp.float32)],
    )(x, c16in, w16)


def _final_body(g0_ref, g1_ref, c16_ref, xk_ref, w0_ref, w1_ref, wc_ref,
                wu_ref, o_ref):
    g0 = g0_ref[...].astype(jnp.float32)
    g1 = g1_ref[...].astype(jnp.float32)
    cnt = g1[:, CNT_COL:CNT_COL + 1]
    num = (jnp.dot(g0, w0_ref[...], preferred_element_type=jnp.float32)
           + jnp.dot(g1, w1_ref[...], preferred_element_type=jnp.float32)
           + cnt * jnp.dot(c16_ref[...], wc_ref[...],
                           preferred_element_type=jnp.float32))
    hn = num / jnp.maximum(cnt, 1.0)
    n = hn.shape[0]
    hn2 = jnp.concatenate(
        [hn, jnp.ones((n, 1), jnp.float32), jnp.zeros((n, 3), jnp.float32)],
        axis=1)
    o_ref[...] = jnp.dot(hn2, wu_ref[...],
                         preferred_element_type=jnp.float32) \
        + xk_ref[...][:, :D_FEAT]


def _tc_final(g0k, g1k, c16k, xk, w0, w1, wc, wu2):
    return pl.pallas_call(
        _final_body,
        grid=(N_KEY // KROW_BLK,),
        in_specs=[
            pl.BlockSpec((KROW_BLK, W), lambda i: (i, 0)),
            pl.BlockSpec((KROW_BLK, W), lambda i: (i, 0)),
            pl.BlockSpec((KROW_BLK, 16), lambda i: (i, 0)),
            pl.BlockSpec((KROW_BLK, XPAD), lambda i: (i, 0)),
            pl.BlockSpec((W, D_FEAT), lambda i: (0, 0)),
            pl.BlockSpec((W, D_FEAT), lambda i: (0, 0)),
            pl.BlockSpec((16, D_FEAT), lambda i: (0, 0)),
            pl.BlockSpec((XPAD, D_FEAT), lambda i: (0, 0)),
        ],
        out_specs=pl.BlockSpec((KROW_BLK, D_FEAT), lambda i: (i, 0)),
        out_shape=jax.ShapeDtypeStruct((N_KEY, D_FEAT), jnp.float32),
    )(g0k, g1k, c16k, xk, w0, w1, wc, wu2)


@functools.cache
def _make_sc_segsum():
    return functools.partial(
        pl.kernel,
        out_type=[jax.ShapeDtypeStruct((N_KEY, W), jnp.bfloat16),
                  jax.ShapeDtypeStruct((N_KEY, W), jnp.bfloat16),
                  jax.ShapeDtypeStruct((N_KEY, XPAD), jnp.float32),
                  jax.ShapeDtypeStruct((N_KEY, 16), jnp.float32)],
        mesh=plsc.VectorSubcoreMesh(core_axis_name="c", subcore_axis_name="s"),
        scratch_types=[
            pltpu.VMEM((2, KB, CH), jnp.int32),
            pltpu.VMEM((2, KB, CH), jnp.int32),
            pltpu.VMEM((4, CH, W), jnp.bfloat16),
            pltpu.VMEM_SHARED((ACC_ROWS, W), jnp.bfloat16),
            pltpu.VMEM((CH,), jnp.int32),
            pltpu.VMEM((CH, W), jnp.bfloat16),
            pltpu.VMEM((32, XPAD), jnp.float32),
            pltpu.VMEM((CH, 16), jnp.float32),
            pltpu.SemaphoreType.DMA,
            pltpu.SemaphoreType.DMA,
            pltpu.SemaphoreType.DMA,
        ],
        compiler_params=pltpu.CompilerParams(use_tc_tiling_on_sc=False),
    )(_sc_segsum_body)


def _sc_segsum_body(a0, a1, src_t, dst_t, zrows, kp_t, xpad, c16,
                    g0k, g1k, xk, c16k,
                    sidx, didx, rows, acc, kidx, kbuf, xbuf, cbuf,
                    sem_g, sem_s, sem_i):
    c = lax.axis_index("c")
    s = lax.axis_index("s")
    NB = NCH // KB
    pltpu.sync_copy(zrows, acc.at[pl.ds(s * ZROWS, ZROWS)])
    plsc.subcore_barrier()

    def idx_fetch(t, bb):
        for k in range(KB):
            pltpu.async_copy(src_t.at[s, t * KB + k], sidx.at[bb, k], sem_i)
            pltpu.async_copy(dst_t.at[s, t * KB + k], didx.at[bb, k], sem_i)

    def idx_drain(t, bb):
        for k in range(KB):
            pltpu.make_async_copy(
                src_t.at[s, t * KB + k], sidx.at[bb, k], sem_i).wait()
            pltpu.make_async_copy(
                dst_t.at[s, t * KB + k], didx.at[bb, k], sem_i).wait()

    def run(a_ref):
        idx_fetch(0, 0)

        def block(t, carry):
            bb = lax.rem(t, 2)
            idx_drain(t, bb)

            @pl.when(t + 1 < NB)
            def _():
                idx_fetch(t + 1, 1 - bb)

            gathers = [
                pltpu.make_async_copy(a_ref.at[sidx.at[bb, k]],
                                      rows.at[k % 4], sem_g)
                for k in range(KB)
            ]
            scatters = [
                pltpu.make_async_copy(rows.at[k % 4],
                                      acc.at[didx.at[bb, k]], sem_s)
                for k in range(KB)
            ]
            gathers[0].start()
            gathers[1].start()
            gathers[2].start()
            for k in range(KB):
                gathers[k].wait()
                if k >= 1:
                    scatters[k - 1].wait()
                if k + 3 < KB:
                    gathers[k + 3].start()
                scatters[k].start(add=True)
            scatters[KB - 1].wait()
            return carry
        lax.fori_loop(0, NB, block, 0)

    @pl.when(c == 0)
    def _():
        run(a0)

    @pl.when(c == 1)
    def _():
        run(a1)

    plsc.subcore_barrier()

    @pl.when(c == 0)
    def _():
        for q in range(2):
            pltpu.sync_copy(kp_t.at[s, q], kidx)
            pltpu.async_copy(acc.at[kidx], kbuf, sem_g).wait()
            pltpu.sync_copy(kbuf, g0k.at[pl.ds(s * KPT + q * CH, CH)])
        pltpu.sync_copy(kp_t.at[s, 0], kidx)
        for i in range(4):
            pltpu.async_copy(
                xpad.at[kidx.at[pl.ds(i * 32, 32)]], xbuf, sem_g).wait()
            pltpu.sync_copy(xbuf, xk.at[pl.ds(s * KPT + i * 32, 32)])

    @pl.when(c == 1)
    def _():
        for q in range(2):
            pltpu.sync_copy(kp_t.at[s, q], kidx)
            pltpu.async_copy(acc.at[kidx], kbuf, sem_g).wait()
            pltpu.sync_copy(kbuf, g1k.at[pl.ds(s * KPT + q * CH, CH)])
            pltpu.async_copy(c16.at[kidx], cbuf, sem_g).wait()
            pltpu.sync_copy(cbuf, c16k.at[pl.ds(s * KPT + q * CH, CH)])
        pltpu.sync_copy(kp_t.at[s, 1], kidx)
        for i in range(4):
            pltpu.async_copy(
                xpad.at[kidx.at[pl.ds(i * 32, 32)]], xbuf, sem_g).wait()
            pltpu.sync_copy(xbuf, xk.at[pl.ds(s * KPT + CH + i * 32, 32)])


def kernel(input_vertex_features, input_vertex_coordinates, keypoint_indices,
           edges, W_off, b_off, W_edge, b_edge, W_upd, b_upd):
    x = input_vertex_features
    coords = input_vertex_coordinates
    f32 = jnp.float32

    ones = jnp.ones((N_NODES, 1), f32)
    c16in = jnp.concatenate(
        [coords + b_off, ones, jnp.zeros((N_NODES, 12), f32)], axis=1)
    w16 = jnp.concatenate([W_off, jnp.zeros((D_FEAT, 13), f32)], axis=1)

    c16, a0, a1, xpad = _tc_offset(x, c16in, w16)

    src = edges[:, 0].astype(jnp.int32)
    dst = edges[:, 1].astype(jnp.int32)
    pad = E_PAD - N_EDGES
    srcp = jnp.concatenate([src, jnp.zeros((pad,), jnp.int32)])
    dstp = jnp.concatenate([dst, jnp.full((pad,), N_NODES, jnp.int32)])
    src_t = srcp.reshape(NS, NCH, CH)
    dst_t = dstp.reshape(NS, NCH, CH)
    zrows = jnp.zeros((ZROWS, W), jnp.bfloat16)
    kp_t = keypoint_indices[:, 0].astype(jnp.int32).reshape(NS, 2, CH)

    g0k, g1k, xk, c16k = _make_sc_segsum()(
        a0, a1, src_t, dst_t, zrows, kp_t, xpad, c16)

    npad = W - (D_FEAT - W) - 4
    w0 = W_edge[:W]
    w1 = jnp.concatenate(
        [W_edge[W:D_FEAT], -W_edge[D_FEAT:D_FEAT + 3], b_edge[None, :],
         jnp.zeros((npad, D_FEAT), f32)], axis=0)
    wc16 = jnp.concatenate([W_edge[D_FEAT:D_FEAT + 3],
                            jnp.zeros((13, D_FEAT), f32)], axis=0)
    wu2 = jnp.concatenate(
        [W_upd, b_upd[None, :], jnp.zeros((3, D_FEAT), f32)], axis=0)

    return _tc_final(g0k, g1k, c16k, xk, w0, w1, wc16, wu2)
